# Initial kernel scaffold; baseline (speedup 1.0000x reference)
import jax
import jax.numpy as jnp
from jax import lax
from jax.experimental import pallas as pl
from jax.experimental.pallas import tpu as pltpu
from jax.experimental.pallas import tpu_sc as plsc

N_NODES = 100000
N_EDGES = 1600000
BATCH = 32

NTILES = 16
NCORES = 2
HB = BATCH // NCORES
K = 128

NPAD = 100032
ROWS_PER_TILE = NPAD // NTILES
EPAD = ((N_EDGES + NTILES * K - 1) // (NTILES * K)) * (NTILES * K)
EDGES_PER_TILE = EPAD // NTILES
CHUNKS = EDGES_PER_TILE // K


def _sc_body(src_hbm, des_hbm, par_hbm, xt_hbm, zer_hbm, out_hbm,
             idx_s, idx_d, idx_sg, idx_dg, parb, vs, vd, os_, od_, acc,
             sem1, sem2):
    c = lax.axis_index("c")
    t = lax.axis_index("s")

    r0 = t * ROWS_PER_TILE
    pltpu.sync_copy(zer_hbm.at[pl.ds(r0, ROWS_PER_TILE)],
                    acc.at[pl.ds(r0, ROWS_PER_TILE)])
    plsc.subcore_barrier()

    off = c * NPAD

    def chunk_body(g, carry):
        base = t * EDGES_PER_TILE + g * K
        pltpu.sync_copy(src_hbm.at[pl.ds(base, K)], idx_s)
        pltpu.sync_copy(des_hbm.at[pl.ds(base, K)], idx_d)
        pltpu.sync_copy(par_hbm.at[pl.ds(base, K)], parb)
        for i in range(K // 16):
            sl = pl.ds(i * 16, 16)
            idx_sg[sl] = idx_s[sl] + off
            idx_dg[sl] = idx_d[sl] + off
        cp1 = pltpu.async_copy(xt_hbm.at[idx_sg], vs, sem1)
        cp2 = pltpu.async_copy(xt_hbm.at[idx_dg], vd, sem2)
        cp1.wait()
        cp2.wait()

        def edge_body(e, _):
            p = parb[e]
            d = vs[e] - vd[e]
            od_[e] = d * p
            os_[e] = d * (-p)
            return 0

        lax.fori_loop(0, K, edge_body, 0)
        pltpu.sync_copy(os_, acc.at[idx_s], add=True)
        pltpu.sync_copy(od_, acc.at[idx_d], add=True)
        return carry

    lax.fori_loop(0, CHUNKS, chunk_body, 0)
    plsc.subcore_barrier()

    pltpu.sync_copy(acc.at[pl.ds(r0, ROWS_PER_TILE)],
                    out_hbm.at[pl.ds(c * NPAD + r0, ROWS_PER_TILE)])


@jax.jit
def kernel(t, x, src, des, param):
    del t
    xp = jnp.pad(x, ((0, 0), (1, NPAD - (N_NODES + 1))))
    aux_t = xp.T
    xt = jnp.concatenate([aux_t[:, :HB], aux_t[:, HB:]], axis=0)

    pad_e = EPAD - N_EDGES
    sp = jnp.pad(src, (0, pad_e))
    dp = jnp.pad(des, (0, pad_e))
    pp = jnp.pad(param, (0, pad_e))
    zer = jnp.zeros((NPAD, HB), jnp.float32)

    mesh = plsc.VectorSubcoreMesh(core_axis_name="c", subcore_axis_name="s")
    fn = pl.kernel(
        _sc_body,
        out_type=jax.ShapeDtypeStruct((2 * NPAD, HB), jnp.float32),
        mesh=mesh,
        scratch_types=[
            pltpu.VMEM((K,), jnp.int32),
            pltpu.VMEM((K,), jnp.int32),
            pltpu.VMEM((K,), jnp.int32),
            pltpu.VMEM((K,), jnp.int32),
            pltpu.VMEM((K,), jnp.float32),
            pltpu.VMEM((K, HB), jnp.float32),
            pltpu.VMEM((K, HB), jnp.float32),
            pltpu.VMEM((K, HB), jnp.float32),
            pltpu.VMEM((K, HB), jnp.float32),
            pltpu.VMEM_SHARED((NPAD, HB), jnp.float32),
            pltpu.SemaphoreType.DMA,
            pltpu.SemaphoreType.DMA,
        ],
    )
    out = fn(sp, dp, pp, xt, zer)
    res_t = jnp.concatenate([out[:NPAD], out[NPAD:]], axis=1)
    return res_t[1:N_NODES + 1, :].T


# speedup vs baseline: 6.6619x; 6.6619x over previous
import jax
import jax.numpy as jnp
from jax import lax
from jax.experimental import pallas as pl
from jax.experimental.pallas import tpu as pltpu
from jax.experimental.pallas import tpu_sc as plsc

N_NODES = 100000
N_EDGES = 1600000
BATCH = 32

NTILES = 16
NCORES = 2
HB = BATCH // NCORES
K = 128

NPAD = 100096
ROWS_PER_TILE = NPAD // NTILES
EPAD = ((N_EDGES + NTILES * K - 1) // (NTILES * K)) * (NTILES * K)
EDGES_PER_TILE = EPAD // NTILES
CHUNKS = EDGES_PER_TILE // K


def _sc_body(src_hbm, des_hbm, par_hbm, xt_hbm, zer_hbm, out_hbm,
             idx_s, idx_d, idx_sg, idx_dg, parb, vs, vd, os_, od_, acc,
             sem1, sem2):
    c = lax.axis_index("c")
    t = lax.axis_index("s")

    r0 = t * ROWS_PER_TILE
    pltpu.sync_copy(zer_hbm.at[pl.ds(r0, ROWS_PER_TILE)],
                    acc.at[pl.ds(r0, ROWS_PER_TILE)])
    plsc.subcore_barrier()

    off = c * NPAD

    def chunk_body(g, carry):
        base = t * EDGES_PER_TILE + g * K
        pltpu.sync_copy(src_hbm.at[pl.ds(base, K)], idx_s)
        pltpu.sync_copy(des_hbm.at[pl.ds(base, K)], idx_d)
        pltpu.sync_copy(par_hbm.at[pl.ds(base, K)], parb)
        for i in range(K // 16):
            sl = pl.ds(i * 16, 16)
            idx_sg[sl] = idx_s[sl] + off
            idx_dg[sl] = idx_d[sl] + off
        cp1 = pltpu.async_copy(xt_hbm.at[idx_sg], vs, sem1)
        cp2 = pltpu.async_copy(xt_hbm.at[idx_dg], vd, sem2)
        cp1.wait()
        cp2.wait()

        def grp_body(q, _):
            e0 = q * 16
            pv = parb[pl.ds(e0, 16)]
            for j in range(16):
                e = e0 + j
                d = vs[e] - vd[e]
                i_cur = d * pv[j]
                od_[e] = i_cur
                os_[e] = -i_cur
            return 0

        lax.fori_loop(0, K // 16, grp_body, 0)
        pltpu.sync_copy(os_, acc.at[idx_s], add=True)
        pltpu.sync_copy(od_, acc.at[idx_d], add=True)
        return carry

    lax.fori_loop(0, CHUNKS, chunk_body, 0)
    plsc.subcore_barrier()

    pltpu.sync_copy(acc.at[pl.ds(r0, ROWS_PER_TILE)],
                    out_hbm.at[pl.ds(c * NPAD + r0, ROWS_PER_TILE)])


@jax.jit
def kernel(t, x, src, des, param):
    del t
    xp = jnp.pad(x, ((0, 0), (1, NPAD - (N_NODES + 1))))
    aux_t = xp.T
    xt = jnp.concatenate([aux_t[:, :HB], aux_t[:, HB:]], axis=0)

    pad_e = EPAD - N_EDGES
    sp = jnp.pad(src, (0, pad_e))
    dp = jnp.pad(des, (0, pad_e))
    pp = jnp.pad(param, (0, pad_e))
    zer = jnp.zeros((NPAD, HB), jnp.float32)

    mesh = plsc.VectorSubcoreMesh(core_axis_name="c", subcore_axis_name="s")
    fn = pl.kernel(
        _sc_body,
        out_type=jax.ShapeDtypeStruct((2 * NPAD, HB), jnp.float32),
        mesh=mesh,
        compiler_params=pltpu.CompilerParams(use_tc_tiling_on_sc=False),
        scratch_types=[
            pltpu.VMEM((K,), jnp.int32),
            pltpu.VMEM((K,), jnp.int32),
            pltpu.VMEM((K,), jnp.int32),
            pltpu.VMEM((K,), jnp.int32),
            pltpu.VMEM((K,), jnp.float32),
            pltpu.VMEM((K, HB), jnp.float32),
            pltpu.VMEM((K, HB), jnp.float32),
            pltpu.VMEM((K, HB), jnp.float32),
            pltpu.VMEM((K, HB), jnp.float32),
            pltpu.VMEM_SHARED((NPAD, HB), jnp.float32),
            pltpu.SemaphoreType.DMA,
            pltpu.SemaphoreType.DMA,
        ],
    )
    out = fn(sp, dp, pp, xt, zer)
    res_t = jnp.concatenate([out[:NPAD], out[NPAD:]], axis=1)
    return res_t[1:N_NODES + 1, :].T


# speedup vs baseline: 18.1839x; 2.7295x over previous
import jax
import jax.numpy as jnp
from jax import lax
from jax.experimental import pallas as pl
from jax.experimental.pallas import tpu as pltpu
from jax.experimental.pallas import tpu_sc as plsc

N_NODES = 100000
N_EDGES = 1600000
BATCH = 32

NTILES = 16
NCORES = 2
HB = BATCH // NCORES
K = 128

NPAD = 100096
ROWS_PER_TILE = NPAD // NTILES
EPAD = ((N_EDGES + NTILES * K - 1) // (NTILES * K)) * (NTILES * K)
EDGES_PER_TILE = EPAD // NTILES
CHUNKS = EDGES_PER_TILE // K
HLOOP = CHUNKS // 2


def _sc_body(src_hbm, des_hbm, par_hbm, xt_hbm, zer_hbm, out_hbm, *scr):
    (rs0, rd0, rp0, sg0, dg0, ss0, dd0, pc0, vs0, vd0, os0, od0,
     rs1, rd1, rp1, sg1, dg1, ss1, dd1, pc1, vs1, vd1, os1, od1,
     acc, semL0, semL1, semG0, semG1, semS0, semS1) = scr
    B0 = (rs0, rd0, rp0, sg0, dg0, ss0, dd0, pc0, vs0, vd0, os0, od0, semL0, semG0, semS0)
    B1 = (rs1, rd1, rp1, sg1, dg1, ss1, dd1, pc1, vs1, vd1, os1, od1, semL1, semG1, semS1)

    c = lax.axis_index("c")
    t = lax.axis_index("s")
    off = c * NPAD

    def base_of(g):
        return t * EDGES_PER_TILE + g * K

    def issue_loads(g, B):
        rs, rd, rp = B[0], B[1], B[2]
        b = base_of(g)
        pltpu.async_copy(src_hbm.at[pl.ds(b, K)], rs, B[12])
        pltpu.async_copy(des_hbm.at[pl.ds(b, K)], rd, B[12])
        pltpu.async_copy(par_hbm.at[pl.ds(b, K)], rp, B[12])

    def wait_loads(g, B):
        rs, rd, rp = B[0], B[1], B[2]
        b = base_of(g)
        pltpu.make_async_copy(src_hbm.at[pl.ds(b, K)], rs, B[12]).wait()
        pltpu.make_async_copy(des_hbm.at[pl.ds(b, K)], rd, B[12]).wait()
        pltpu.make_async_copy(par_hbm.at[pl.ds(b, K)], rp, B[12]).wait()

    def offset_pass(B):
        rs, rd, rp, sg, dg, ss, dd, pc = B[0], B[1], B[2], B[3], B[4], B[5], B[6], B[7]
        for i in range(K // 16):
            sl = pl.ds(i * 16, 16)
            s_ = rs[sl]
            d_ = rd[sl]
            sg[sl] = s_ + off
            dg[sl] = d_ + off
            ss[sl] = s_
            dd[sl] = d_
            pc[sl] = rp[sl]

    def issue_gathers(B):
        sg, dg, vs, vd = B[3], B[4], B[8], B[9]
        pltpu.async_copy(xt_hbm.at[sg], vs, B[13])
        pltpu.async_copy(xt_hbm.at[dg], vd, B[13])

    def wait_gathers(B):
        sg, dg, vs, vd = B[3], B[4], B[8], B[9]
        pltpu.make_async_copy(xt_hbm.at[sg], vs, B[13]).wait()
        pltpu.make_async_copy(xt_hbm.at[dg], vd, B[13]).wait()

    def compute(B):
        pc, vs, vd, os_, od_ = B[7], B[8], B[9], B[10], B[11]

        def grp_body(q, _):
            e0 = q * 16
            pv = pc[pl.ds(e0, 16)]
            for j in range(16):
                e = e0 + j
                d = vs[e] - vd[e]
                i_cur = d * pv[j]
                od_[e] = i_cur
                os_[e] = -i_cur
            return 0

        lax.fori_loop(0, K // 16, grp_body, 0)

    def issue_scatters(B):
        ss, dd, os_, od_ = B[5], B[6], B[10], B[11]
        pltpu.async_copy(os_, acc.at[ss], B[14], add=True)
        pltpu.async_copy(od_, acc.at[dd], B[14], add=True)

    def wait_scatters(B):
        ss, dd, os_, od_ = B[5], B[6], B[10], B[11]
        pltpu.make_async_copy(os_, acc.at[ss], B[14]).wait()
        pltpu.make_async_copy(od_, acc.at[dd], B[14]).wait()

    r0 = t * ROWS_PER_TILE
    pltpu.sync_copy(zer_hbm.at[pl.ds(r0, ROWS_PER_TILE)],
                    acc.at[pl.ds(r0, ROWS_PER_TILE)])
    plsc.subcore_barrier()

    issue_loads(0, B0)
    issue_loads(1, B1)
    wait_loads(0, B0)
    offset_pass(B0)
    issue_gathers(B0)

    def h_body(h, carry):
        wait_loads(2 * h + 1, B1)

        @pl.when(h >= 1)
        def _():
            wait_scatters(B1)

        offset_pass(B1)
        issue_gathers(B1)

        @pl.when(h <= HLOOP - 2)
        def _():
            issue_loads(2 * h + 2, B0)

        wait_gathers(B0)
        compute(B0)
        issue_scatters(B0)

        @pl.when(h <= HLOOP - 2)
        def _():
            wait_loads(2 * h + 2, B0)
            wait_scatters(B0)
            offset_pass(B0)
            issue_gathers(B0)
            issue_loads(2 * h + 3, B1)

        wait_gathers(B1)
        compute(B1)
        issue_scatters(B1)
        return carry

    lax.fori_loop(0, HLOOP, h_body, 0)

    wait_scatters(B0)
    wait_scatters(B1)
    plsc.subcore_barrier()

    pltpu.sync_copy(acc.at[pl.ds(r0, ROWS_PER_TILE)],
                    out_hbm.at[pl.ds(c * NPAD + r0, ROWS_PER_TILE)])


@jax.jit
def kernel(t, x, src, des, param):
    del t
    xp = jnp.pad(x, ((0, 0), (1, NPAD - (N_NODES + 1))))
    aux_t = xp.T
    xt = jnp.concatenate([aux_t[:, :HB], aux_t[:, HB:]], axis=0)

    pad_e = EPAD - N_EDGES
    sp = jnp.pad(src, (0, pad_e))
    dp = jnp.pad(des, (0, pad_e))
    pp = jnp.pad(param, (0, pad_e))
    zer = jnp.zeros((NPAD, HB), jnp.float32)

    def buf_set():
        return [
            pltpu.VMEM((K,), jnp.int32),
            pltpu.VMEM((K,), jnp.int32),
            pltpu.VMEM((K,), jnp.float32),
            pltpu.VMEM((K,), jnp.int32),
            pltpu.VMEM((K,), jnp.int32),
            pltpu.VMEM((K,), jnp.int32),
            pltpu.VMEM((K,), jnp.int32),
            pltpu.VMEM((K,), jnp.float32),
            pltpu.VMEM((K, HB), jnp.float32),
            pltpu.VMEM((K, HB), jnp.float32),
            pltpu.VMEM((K, HB), jnp.float32),
            pltpu.VMEM((K, HB), jnp.float32),
        ]

    mesh = plsc.VectorSubcoreMesh(core_axis_name="c", subcore_axis_name="s")
    fn = pl.kernel(
        _sc_body,
        out_type=jax.ShapeDtypeStruct((2 * NPAD, HB), jnp.float32),
        mesh=mesh,
        compiler_params=pltpu.CompilerParams(use_tc_tiling_on_sc=False),
        scratch_types=buf_set() + buf_set() + [
            pltpu.VMEM_SHARED((NPAD, HB), jnp.float32),
            pltpu.SemaphoreType.DMA,
            pltpu.SemaphoreType.DMA,
            pltpu.SemaphoreType.DMA,
            pltpu.SemaphoreType.DMA,
            pltpu.SemaphoreType.DMA,
            pltpu.SemaphoreType.DMA,
        ],
    )
    out = fn(sp, dp, pp, xt, zer)
    res_t = jnp.concatenate([out[:NPAD], out[NPAD:]], axis=1)
    return res_t[1:N_NODES + 1, :].T


# speedup vs baseline: 18.8225x; 1.0351x over previous
import jax
import jax.numpy as jnp
from jax import lax
from jax.experimental import pallas as pl
from jax.experimental.pallas import tpu as pltpu
from jax.experimental.pallas import tpu_sc as plsc

N_NODES = 100000
N_EDGES = 1600000
BATCH = 32

NTILES = 16
NCORES = 2
HB = BATCH // NCORES
K = 128

NPAD = 100096
ROWS_PER_TILE = NPAD // NTILES
TOT_CHUNKS = N_EDGES // K
BASE_CHUNKS = TOT_CHUNKS // NTILES
EXTRA_TILES = TOT_CHUNKS - BASE_CHUNKS * NTILES
MAIN_CHUNKS = BASE_CHUNKS - 1
HLOOP = MAIN_CHUNKS // 2


def _sc_body(src_hbm, des_hbm, par_hbm, xt_hbm, zer_hbm, out_hbm, *scr):
    (rs0, rd0, rp0, sg0, dg0, ss0, dd0, pc0, vs0, vd0, os0, od0,
     rs1, rd1, rp1, sg1, dg1, ss1, dd1, pc1, vs1, vd1, os1, od1,
     acc, semL0, semL1, semG0, semG1, semS0, semS1) = scr
    B0 = (rs0, rd0, rp0, sg0, dg0, ss0, dd0, pc0, vs0, vd0, os0, od0, semL0, semG0, semS0)
    B1 = (rs1, rd1, rp1, sg1, dg1, ss1, dd1, pc1, vs1, vd1, os1, od1, semL1, semG1, semS1)

    c = lax.axis_index("c")
    t = lax.axis_index("s")
    off = c * NPAD
    tile_base = t * BASE_CHUNKS * K

    def issue_loads(b, B):
        pltpu.async_copy(src_hbm.at[pl.ds(b, K)], B[0], B[12])
        pltpu.async_copy(des_hbm.at[pl.ds(b, K)], B[1], B[12])
        pltpu.async_copy(par_hbm.at[pl.ds(b, K)], B[2], B[12])

    def wait_loads(b, B):
        pltpu.make_async_copy(src_hbm.at[pl.ds(b, K)], B[0], B[12]).wait()
        pltpu.make_async_copy(des_hbm.at[pl.ds(b, K)], B[1], B[12]).wait()
        pltpu.make_async_copy(par_hbm.at[pl.ds(b, K)], B[2], B[12]).wait()

    def offset_pass(B):
        rs, rd, rp, sg, dg, ss, dd, pc = B[0], B[1], B[2], B[3], B[4], B[5], B[6], B[7]
        for i in range(K // 16):
            sl = pl.ds(i * 16, 16)
            s_ = rs[sl]
            d_ = rd[sl]
            sg[sl] = s_ + off
            dg[sl] = d_ + off
            ss[sl] = s_
            dd[sl] = d_
            pc[sl] = rp[sl]

    def issue_gathers(B):
        pltpu.async_copy(xt_hbm.at[B[3]], B[8], B[13])
        pltpu.async_copy(xt_hbm.at[B[4]], B[9], B[13])

    def wait_gathers(B):
        pltpu.make_async_copy(xt_hbm.at[B[3]], B[8], B[13]).wait()
        pltpu.make_async_copy(xt_hbm.at[B[4]], B[9], B[13]).wait()

    def compute(B):
        pc, vs, vd, os_, od_ = B[7], B[8], B[9], B[10], B[11]

        def grp_body(q, _):
            e0 = q * 16
            pv = pc[pl.ds(e0, 16)]
            for j in range(16):
                e = e0 + j
                d = vs[e] - vd[e]
                i_cur = d * pv[j]
                od_[e] = i_cur
                os_[e] = -i_cur
            return 0

        lax.fori_loop(0, K // 16, grp_body, 0)

    def issue_scatters(B):
        pltpu.async_copy(B[10], acc.at[B[5]], B[14], add=True)
        pltpu.async_copy(B[11], acc.at[B[6]], B[14], add=True)

    def wait_scatters(B):
        pltpu.make_async_copy(B[10], acc.at[B[5]], B[14]).wait()
        pltpu.make_async_copy(B[11], acc.at[B[6]], B[14]).wait()

    r0 = t * ROWS_PER_TILE
    pltpu.sync_copy(zer_hbm, acc.at[pl.ds(r0, ROWS_PER_TILE)])
    plsc.subcore_barrier()

    issue_loads(tile_base, B0)
    issue_loads(tile_base + K, B1)
    wait_loads(tile_base, B0)
    offset_pass(B0)
    issue_gathers(B0)

    def h_body(h, carry):
        wait_loads(tile_base + (2 * h + 1) * K, B1)

        @pl.when(h >= 1)
        def _():
            wait_scatters(B1)

        offset_pass(B1)
        issue_gathers(B1)

        @pl.when(h <= HLOOP - 2)
        def _():
            issue_loads(tile_base + (2 * h + 2) * K, B0)

        wait_gathers(B0)
        compute(B0)
        issue_scatters(B0)

        @pl.when(h <= HLOOP - 2)
        def _():
            wait_loads(tile_base + (2 * h + 2) * K, B0)
            wait_scatters(B0)
            offset_pass(B0)
            issue_gathers(B0)
            issue_loads(tile_base + (2 * h + 3) * K, B1)

        wait_gathers(B1)
        compute(B1)
        issue_scatters(B1)
        return carry

    lax.fori_loop(0, HLOOP, h_body, 0)

    wait_scatters(B0)
    wait_scatters(B1)

    def sync_chunk(b, B):
        issue_loads(b, B)
        wait_loads(b, B)
        offset_pass(B)
        issue_gathers(B)
        wait_gathers(B)
        compute(B)
        issue_scatters(B)
        wait_scatters(B)

    sync_chunk(tile_base + MAIN_CHUNKS * K, B0)

    @pl.when(t < EXTRA_TILES)
    def _():
        sync_chunk((BASE_CHUNKS * NTILES + t) * K, B1)

    plsc.subcore_barrier()

    pltpu.sync_copy(acc.at[pl.ds(r0, ROWS_PER_TILE)],
                    out_hbm.at[pl.ds(c * NPAD + r0, ROWS_PER_TILE)])


@jax.jit
def kernel(t, x, src, des, param):
    del t
    xp = jnp.pad(x, ((0, 0), (1, NPAD - (N_NODES + 1))))
    xt = jnp.transpose(xp.reshape(NCORES, HB, NPAD), (0, 2, 1)).reshape(
        NCORES * NPAD, HB)
    zer = jnp.zeros((ROWS_PER_TILE, HB), jnp.float32)

    def buf_set():
        return [
            pltpu.VMEM((K,), jnp.int32),
            pltpu.VMEM((K,), jnp.int32),
            pltpu.VMEM((K,), jnp.float32),
            pltpu.VMEM((K,), jnp.int32),
            pltpu.VMEM((K,), jnp.int32),
            pltpu.VMEM((K,), jnp.int32),
            pltpu.VMEM((K,), jnp.int32),
            pltpu.VMEM((K,), jnp.float32),
            pltpu.VMEM((K, HB), jnp.float32),
            pltpu.VMEM((K, HB), jnp.float32),
            pltpu.VMEM((K, HB), jnp.float32),
            pltpu.VMEM((K, HB), jnp.float32),
        ]

    mesh = plsc.VectorSubcoreMesh(core_axis_name="c", subcore_axis_name="s")
    fn = pl.kernel(
        _sc_body,
        out_type=jax.ShapeDtypeStruct((NCORES * NPAD, HB), jnp.float32),
        mesh=mesh,
        compiler_params=pltpu.CompilerParams(use_tc_tiling_on_sc=False),
        scratch_types=buf_set() + buf_set() + [
            pltpu.VMEM_SHARED((NPAD, HB), jnp.float32),
            pltpu.SemaphoreType.DMA,
            pltpu.SemaphoreType.DMA,
            pltpu.SemaphoreType.DMA,
            pltpu.SemaphoreType.DMA,
            pltpu.SemaphoreType.DMA,
            pltpu.SemaphoreType.DMA,
        ],
    )
    out = fn(src, des, param, xt, zer)
    res_t = jnp.concatenate([out[:NPAD], out[NPAD:]], axis=1)
    return res_t[1:N_NODES + 1, :].T


# speedup vs baseline: 21.2119x; 1.1269x over previous
import jax
import jax.numpy as jnp
from jax import lax
from jax.experimental import pallas as pl
from jax.experimental.pallas import tpu as pltpu
from jax.experimental.pallas import tpu_sc as plsc

N_NODES = 100000
N_EDGES = 1600000
BATCH = 32

NTILES = 16
NCORES = 2
HB = BATCH // NCORES
K = 128

NPAD = 100096
ROWS_PER_TILE = NPAD // NTILES
TOT_CHUNKS = N_EDGES // K
BASE_CHUNKS = TOT_CHUNKS // NTILES
EXTRA_TILES = TOT_CHUNKS - BASE_CHUNKS * NTILES
MAIN_CHUNKS = BASE_CHUNKS - 1
HLOOP = MAIN_CHUNKS // 2


def _sc_body(src_hbm, des_hbm, par_hbm, xt_hbm, zer_hbm, out_hbm, *scr):
    (rs0, rd0, rp0, sg0, dg0, ss0, dd0, pc0, vs0, vd0, os0, od0,
     rs1, rd1, rp1, sg1, dg1, ss1, dd1, pc1, vs1, vd1, os1, od1,
     acc, semL0, semL1, semG0, semG1, semS0, semS1) = scr
    B0 = (rs0, rd0, rp0, sg0, dg0, ss0, dd0, pc0, vs0, vd0, os0, od0, semL0, semG0, semS0)
    B1 = (rs1, rd1, rp1, sg1, dg1, ss1, dd1, pc1, vs1, vd1, os1, od1, semL1, semG1, semS1)

    c = lax.axis_index("c")
    t = lax.axis_index("s")
    off = c * NPAD
    tile_base = t * BASE_CHUNKS * K

    def issue_loads(b, B):
        pltpu.async_copy(src_hbm.at[pl.ds(b, K)], B[0], B[12])
        pltpu.async_copy(des_hbm.at[pl.ds(b, K)], B[1], B[12])
        pltpu.async_copy(par_hbm.at[pl.ds(b, K)], B[2], B[12])

    def wait_loads(b, B):
        pltpu.make_async_copy(src_hbm.at[pl.ds(b, K)], B[0], B[12]).wait()
        pltpu.make_async_copy(des_hbm.at[pl.ds(b, K)], B[1], B[12]).wait()
        pltpu.make_async_copy(par_hbm.at[pl.ds(b, K)], B[2], B[12]).wait()

    def offset_pass(B):
        rs, rd, rp, sg, dg, ss, dd, pc = B[0], B[1], B[2], B[3], B[4], B[5], B[6], B[7]
        for i in range(K // 16):
            sl = pl.ds(i * 16, 16)
            s_ = rs[sl]
            d_ = rd[sl]
            sg[sl] = s_ + off
            dg[sl] = d_ + off
            ss[sl] = s_
            dd[sl] = d_
            pc[sl] = rp[sl]

    def issue_gathers(B):
        pltpu.async_copy(xt_hbm.at[B[3]], B[8], B[13])
        pltpu.async_copy(xt_hbm.at[B[4]], B[9], B[13])

    def wait_gathers(B):
        pltpu.make_async_copy(xt_hbm.at[B[3]], B[8], B[13]).wait()
        pltpu.make_async_copy(xt_hbm.at[B[4]], B[9], B[13]).wait()

    def compute(B):
        pc, vs, vd, os_, od_ = B[7], B[8], B[9], B[10], B[11]

        def grp_body(q, _):
            e0 = q * 16
            pv = pc[pl.ds(e0, 16)]
            for j in range(16):
                e = e0 + j
                d = vs[e] - vd[e]
                i_cur = d * pv[j]
                od_[e] = i_cur
                os_[e] = -i_cur
            return 0

        lax.fori_loop(0, K // 16, grp_body, 0)

    def issue_scatters(B):
        pltpu.async_copy(B[10], acc.at[B[5]], B[14], add=True)
        pltpu.async_copy(B[11], acc.at[B[6]], B[14], add=True)

    def wait_scatters(B):
        pltpu.make_async_copy(B[10], acc.at[B[5]], B[14]).wait()
        pltpu.make_async_copy(B[11], acc.at[B[6]], B[14]).wait()

    r0 = t * ROWS_PER_TILE
    pltpu.sync_copy(zer_hbm, acc.at[pl.ds(r0, ROWS_PER_TILE)])
    plsc.subcore_barrier()

    issue_loads(tile_base, B0)
    issue_loads(tile_base + K, B1)
    wait_loads(tile_base, B0)
    offset_pass(B0)
    issue_gathers(B0)

    def h_body(h, carry):
        wait_loads(tile_base + (2 * h + 1) * K, B1)

        @pl.when(h >= 1)
        def _():
            wait_scatters(B1)

        offset_pass(B1)
        issue_gathers(B1)

        @pl.when(h <= HLOOP - 2)
        def _():
            issue_loads(tile_base + (2 * h + 2) * K, B0)

        wait_gathers(B0)
        compute(B0)
        issue_scatters(B0)

        @pl.when(h <= HLOOP - 2)
        def _():
            wait_loads(tile_base + (2 * h + 2) * K, B0)
            wait_scatters(B0)
            offset_pass(B0)
            issue_gathers(B0)
            issue_loads(tile_base + (2 * h + 3) * K, B1)

        wait_gathers(B1)
        compute(B1)
        issue_scatters(B1)
        return carry

    lax.fori_loop(0, HLOOP, h_body, 0)

    wait_scatters(B0)
    wait_scatters(B1)

    def sync_chunk(b, B):
        issue_loads(b, B)
        wait_loads(b, B)
        offset_pass(B)
        issue_gathers(B)
        wait_gathers(B)
        compute(B)
        issue_scatters(B)
        wait_scatters(B)

    sync_chunk(tile_base + MAIN_CHUNKS * K, B0)

    @pl.when(t < EXTRA_TILES)
    def _():
        sync_chunk((BASE_CHUNKS * NTILES + t) * K, B1)

    plsc.subcore_barrier()

    pltpu.sync_copy(acc.at[pl.ds(r0, ROWS_PER_TILE)],
                    out_hbm.at[pl.ds(c * NPAD + r0, ROWS_PER_TILE)])


@jax.jit
def kernel(t, x, src, des, param):
    del t
    xp = jnp.pad(x, ((0, 0), (1, NPAD - (N_NODES + 1))))
    xt = jnp.transpose(xp.reshape(NCORES, HB, NPAD), (0, 2, 1)).reshape(
        NCORES * NPAD, HB)
    zer = jnp.zeros((ROWS_PER_TILE, HB), jnp.float32)

    def buf_set():
        return [
            pltpu.VMEM((K,), jnp.int32),
            pltpu.VMEM((K,), jnp.int32),
            pltpu.VMEM((K,), jnp.float32),
            pltpu.VMEM((K,), jnp.int32),
            pltpu.VMEM((K,), jnp.int32),
            pltpu.VMEM((K,), jnp.int32),
            pltpu.VMEM((K,), jnp.int32),
            pltpu.VMEM((K,), jnp.float32),
            pltpu.VMEM((K, HB), jnp.float32),
            pltpu.VMEM((K, HB), jnp.float32),
            pltpu.VMEM((K, HB), jnp.float32),
            pltpu.VMEM((K, HB), jnp.float32),
        ]

    mesh = plsc.VectorSubcoreMesh(core_axis_name="c", subcore_axis_name="s")
    fn = pl.kernel(
        _sc_body,
        out_type=jax.ShapeDtypeStruct((NCORES * NPAD, HB), jnp.float32),
        mesh=mesh,
        compiler_params=pltpu.CompilerParams(use_tc_tiling_on_sc=False),
        scratch_types=buf_set() + buf_set() + [
            pltpu.VMEM_SHARED((NPAD, HB), jnp.float32),
            pltpu.SemaphoreType.DMA,
            pltpu.SemaphoreType.DMA,
            pltpu.SemaphoreType.DMA,
            pltpu.SemaphoreType.DMA,
            pltpu.SemaphoreType.DMA,
            pltpu.SemaphoreType.DMA,
        ],
    )
    out = fn(src, des, param, xt, zer)
    res = jnp.transpose(out.reshape(NCORES, NPAD, HB), (0, 2, 1)).reshape(
        BATCH, NPAD)
    return res[:, 1:N_NODES + 1]
